# Initial kernel scaffold; baseline (speedup 1.0000x reference)
import jax
import jax.numpy as jnp
from jax.experimental import pallas as pl


def kernel(x, x_0):
    raise NotImplementedError("write your pallas kernel here")


# speedup vs baseline: 1.8315x; 1.8315x over previous
import functools

import jax
import jax.numpy as jnp
from jax import lax
from jax.experimental import pallas as pl
from jax.experimental.pallas import tpu as pltpu
from jax.experimental.pallas import tpu_sc as plsc

D = 64
NC = 2
NS = 16
NW = NC * NS
CHUNK = 512


@functools.cache
def _gather_fn(B: int):
  b_per_w = B // NW
  n_chunks = b_per_w // CHUNK
  mesh = plsc.VectorSubcoreMesh(core_axis_name="c", subcore_axis_name="s")

  @functools.partial(
      pl.kernel,
      out_type=jax.ShapeDtypeStruct((B, D), jnp.float32),
      mesh=mesh,
      compiler_params=pltpu.CompilerParams(use_tc_tiling_on_sc=False),
      scratch_types=[
          pltpu.VMEM((b_per_w,), jnp.int32),
          pltpu.VMEM((CHUNK, D), jnp.float32),
          pltpu.SemaphoreType.DMA,
      ],
  )
  def gather(table_hbm, idx_hbm, out_hbm, idx_v, buf, gsem):
    wid = lax.axis_index("s") * NC + lax.axis_index("c")
    base = wid * b_per_w
    pltpu.sync_copy(idx_hbm.at[pl.ds(base, b_per_w)], idx_v)

    def body(c, carry):
      pltpu.async_copy(
          table_hbm.at[idx_v.at[pl.ds(c * CHUNK, CHUNK)]],
          buf,
          gsem,
      ).wait()
      pltpu.sync_copy(buf, out_hbm.at[pl.ds(base + c * CHUNK, CHUNK)])
      return carry

    lax.fori_loop(0, n_chunks, body, 0)

  return gather


def kernel(x, x_0):
  B, H = x.shape
  flat = x.reshape(B * H)
  out = _gather_fn(B * H)(x_0, flat)
  return out.reshape(B, H, D)


# speedup vs baseline: 1.8710x; 1.0215x over previous
import functools

import jax
import jax.numpy as jnp
from jax import lax
from jax.experimental import pallas as pl
from jax.experimental.pallas import tpu as pltpu
from jax.experimental.pallas import tpu_sc as plsc

D = 64
NC = 2
NS = 16
NW = NC * NS
CHUNK = 512


@functools.cache
def _gather_fn(B: int):
  b_per_w = B // NW
  n_chunks = b_per_w // CHUNK
  mesh = plsc.VectorSubcoreMesh(core_axis_name="c", subcore_axis_name="s")

  @functools.partial(
      pl.kernel,
      out_type=jax.ShapeDtypeStruct((B, D), jnp.float32),
      mesh=mesh,
      compiler_params=pltpu.CompilerParams(use_tc_tiling_on_sc=False),
      scratch_types=[
          pltpu.VMEM((b_per_w,), jnp.int32),
          pltpu.VMEM((2, CHUNK, D), jnp.float32),
          pltpu.SemaphoreType.DMA,
          pltpu.SemaphoreType.DMA,
      ],
  )
  def gather(table_hbm, idx_hbm, out_hbm, idx_v, bufs, gsem, wsem):
    wid = lax.axis_index("s") * NC + lax.axis_index("c")
    base = wid * b_per_w
    pltpu.sync_copy(idx_hbm.at[pl.ds(base, b_per_w)], idx_v)

    def gdesc(c, slot):
      return pltpu.make_async_copy(
          table_hbm.at[idx_v.at[pl.ds(c * CHUNK, CHUNK)]],
          bufs.at[slot],
          gsem,
      )

    def wdesc(c, slot):
      return pltpu.make_async_copy(
          bufs.at[slot],
          out_hbm.at[pl.ds(base + c * CHUNK, CHUNK)],
          wsem,
      )

    gdesc(0, 0).start()

    def body(c, carry):
      slot = c % 2
      gdesc(c, slot).wait()

      @pl.when(c + 1 < n_chunks)
      def _():
        @pl.when(c >= 1)
        def _():
          wdesc(c - 1, 1 - slot).wait()

        gdesc(c + 1, 1 - slot).start()

      wdesc(c, slot).start()
      return carry

    lax.fori_loop(0, n_chunks, body, 0)
    wdesc(n_chunks - 2, n_chunks % 2).wait()
    wdesc(n_chunks - 1, (n_chunks - 1) % 2).wait()

  return gather


def kernel(x, x_0):
  B, H = x.shape
  flat = x.reshape(B * H)
  out = _gather_fn(B * H)(x_0, flat)
  return out.reshape(B, H, D)
